# baseline (speedup 1.0000x reference)
import functools
import jax
import jax.numpy as jnp
from jax import lax
from jax.experimental import pallas as pl
from jax.experimental.pallas import tpu as pltpu
from jax.experimental.pallas import tpu_sc as plsc

HIDDEN = 1024
NC, NS = 2, 16
NW = NC * NS
CHUNK_ROWS = 32
CHUNK = CHUNK_ROWS * HIDDEN


def kernel(x, table):
    b, n, h = x.shape
    rows = b * n
    rpw = rows // NW
    nchunk = rpw // CHUNK_ROWS

    xf = x.reshape(-1)
    tf = table.reshape(-1)

    mesh = plsc.VectorSubcoreMesh(
        core_axis_name="c", subcore_axis_name="s",
        num_cores=NC, num_subcores=NS)

    @functools.partial(
        pl.kernel,
        out_type=jax.ShapeDtypeStruct((rows * h,), jnp.float32),
        mesh=mesh,
        scratch_types=[
            pltpu.VMEM((CHUNK,), jnp.float32),
            pltpu.VMEM((CHUNK,), jnp.float32),
            pltpu.SemaphoreType.DMA,
            pltpu.SemaphoreType.DMA,
        ],
    )
    def sc_add(x_hbm, t_hbm, o_hbm, xv, tv, semx, semt):
        wid = lax.axis_index("s") * NC + lax.axis_index("c")
        row0 = wid * rpw
        xbase = row0 * h
        tbase = lax.rem(row0, n) * h

        def chunk_body(ci, carry):
            xoff = pl.multiple_of(xbase + ci * CHUNK, CHUNK)
            toff = pl.multiple_of(tbase + ci * CHUNK, CHUNK)
            cx = pltpu.async_copy(x_hbm.at[pl.ds(xoff, CHUNK)], xv, semx)
            ct = pltpu.async_copy(t_hbm.at[pl.ds(toff, CHUNK)], tv, semt)
            cx.wait()
            ct.wait()

            @plsc.parallel_loop(0, CHUNK, step=16, unroll=8)
            def _add(g):
                plsc.addupdate(xv.at[pl.ds(g, 16)], tv[pl.ds(g, 16)])

            pltpu.sync_copy(xv, o_hbm.at[pl.ds(xoff, CHUNK)])
            return carry

        lax.fori_loop(0, nchunk, chunk_body, 0)

    out = sc_add(xf, tf)
    return out.reshape(b, n, h)


# speedup vs baseline: 1.0393x; 1.0393x over previous
import functools
import jax
import jax.numpy as jnp
from jax import lax
from jax.experimental import pallas as pl
from jax.experimental.pallas import tpu as pltpu
from jax.experimental.pallas import tpu_sc as plsc

HIDDEN = 1024
NC, NS = 2, 16
NW = NC * NS
CHUNK_ROWS = 16
CHUNK = CHUNK_ROWS * HIDDEN
UNROLL = 8


def kernel(x, table):
    b, n, h = x.shape
    rows = b * n
    rpw = rows // NW
    nchunk = rpw // CHUNK_ROWS

    xf = x.reshape(-1)
    tf = table.reshape(-1)

    mesh = plsc.VectorSubcoreMesh(
        core_axis_name="c", subcore_axis_name="s",
        num_cores=NC, num_subcores=NS)

    @functools.partial(
        pl.kernel,
        out_type=jax.ShapeDtypeStruct((rows * h,), jnp.float32),
        mesh=mesh,
        scratch_types=[
            pltpu.VMEM((2, CHUNK), jnp.float32),
            pltpu.VMEM((2, CHUNK), jnp.float32),
            pltpu.SemaphoreType.DMA,
            pltpu.SemaphoreType.DMA,
            pltpu.SemaphoreType.DMA,
            pltpu.SemaphoreType.DMA,
            pltpu.SemaphoreType.DMA,
            pltpu.SemaphoreType.DMA,
        ],
    )
    def sc_add(x_hbm, t_hbm, o_hbm, xv, tv, sx0, sx1, st0, st1, so0, so1):
        wid = lax.axis_index("s") * NC + lax.axis_index("c")
        xbase = wid * rpw * h
        tbase = lax.rem(wid * rpw, n) * h
        semx = (sx0, sx1)
        semt = (st0, st1)
        semo = (so0, so1)

        def start_in(c):
            buf = c % 2
            xoff = pl.multiple_of(xbase + c * CHUNK, CHUNK)
            toff = pl.multiple_of(tbase + c * CHUNK, CHUNK)
            hx = pltpu.async_copy(
                x_hbm.at[pl.ds(xoff, CHUNK)], xv.at[buf], semx[buf])
            ht = pltpu.async_copy(
                t_hbm.at[pl.ds(toff, CHUNK)], tv.at[buf], semt[buf])
            return hx, ht

        in_flight = {0: start_in(0)}
        out_flight = {}
        for c in range(nchunk):
            buf = c % 2
            hx, ht = in_flight.pop(c)
            hx.wait()
            ht.wait()
            if c + 1 < nchunk:
                if c - 1 in out_flight:
                    out_flight.pop(c - 1).wait()
                in_flight[c + 1] = start_in(c + 1)

            @plsc.parallel_loop(0, CHUNK, step=16, unroll=UNROLL)
            def _add(g):
                plsc.addupdate(xv.at[buf, pl.ds(g, 16)], tv[buf, pl.ds(g, 16)])

            xoff = pl.multiple_of(xbase + c * CHUNK, CHUNK)
            out_flight[c] = pltpu.async_copy(
                xv.at[buf], o_hbm.at[pl.ds(xoff, CHUNK)], semo[buf])
        for c in sorted(out_flight):
            out_flight.pop(c).wait()

    out = sc_add(xf, tf)
    return out.reshape(b, n, h)


# speedup vs baseline: 1.0454x; 1.0059x over previous
import functools
import jax
import jax.numpy as jnp
from jax import lax
from jax.experimental import pallas as pl
from jax.experimental.pallas import tpu as pltpu
from jax.experimental.pallas import tpu_sc as plsc

HIDDEN = 1024
NC, NS = 2, 16
NW = NC * NS
CHUNK_ROWS = 16
CHUNK = CHUNK_ROWS * HIDDEN
UNROLL = 16


def kernel(x, table):
    b, n, h = x.shape
    rows = b * n
    rpw = rows // NW
    nchunk = rpw // CHUNK_ROWS

    xf = x.reshape(-1)
    tf = table.reshape(-1)

    mesh = plsc.VectorSubcoreMesh(
        core_axis_name="c", subcore_axis_name="s",
        num_cores=NC, num_subcores=NS)

    @functools.partial(
        pl.kernel,
        out_type=jax.ShapeDtypeStruct((rows * h,), jnp.float32),
        mesh=mesh,
        scratch_types=[
            pltpu.VMEM((2, CHUNK), jnp.float32),
            pltpu.VMEM((2, CHUNK), jnp.float32),
            pltpu.VMEM((2, CHUNK), jnp.float32),
            pltpu.SemaphoreType.DMA,
            pltpu.SemaphoreType.DMA,
            pltpu.SemaphoreType.DMA,
            pltpu.SemaphoreType.DMA,
            pltpu.SemaphoreType.DMA,
            pltpu.SemaphoreType.DMA,
        ],
    )
    def sc_add(x_hbm, t_hbm, o_hbm, xv, tv, ov,
               sx0, sx1, st0, st1, so0, so1):
        wid = lax.axis_index("s") * NC + lax.axis_index("c")
        xbase = wid * rpw * h
        tbase = lax.rem(wid * rpw, n) * h
        semx = (sx0, sx1)
        semt = (st0, st1)
        semo = (so0, so1)

        def start_in(c):
            buf = c % 2
            xoff = pl.multiple_of(xbase + c * CHUNK, CHUNK)
            toff = pl.multiple_of(tbase + c * CHUNK, CHUNK)
            hx = pltpu.async_copy(
                x_hbm.at[pl.ds(xoff, CHUNK)], xv.at[buf], semx[buf])
            ht = pltpu.async_copy(
                t_hbm.at[pl.ds(toff, CHUNK)], tv.at[buf], semt[buf])
            return hx, ht

        in_flight = {0: start_in(0)}
        out_flight = {}
        for c in range(nchunk):
            buf = c % 2
            hx, ht = in_flight.pop(c)
            hx.wait()
            ht.wait()
            if c + 1 < nchunk:
                in_flight[c + 1] = start_in(c + 1)
            if c - 2 in out_flight:
                out_flight.pop(c - 2).wait()

            @plsc.parallel_loop(0, CHUNK, step=16, unroll=UNROLL)
            def _add(g):
                ov[buf, pl.ds(g, 16)] = (
                    xv[buf, pl.ds(g, 16)] + tv[buf, pl.ds(g, 16)])

            xoff = pl.multiple_of(xbase + c * CHUNK, CHUNK)
            out_flight[c] = pltpu.async_copy(
                ov.at[buf], o_hbm.at[pl.ds(xoff, CHUNK)], semo[buf])
        for c in sorted(out_flight):
            out_flight.pop(c).wait()

    out = sc_add(xf, tf)
    return out.reshape(b, n, h)


# speedup vs baseline: 3.0410x; 2.9090x over previous
import functools
import jax
import jax.numpy as jnp
from jax import lax
from jax.experimental import pallas as pl
from jax.experimental.pallas import tpu as pltpu
from jax.experimental.pallas import tpu_sc as plsc

HIDDEN = 1024
NC, NS = 2, 16
NW = NC * NS
CHUNK_ROWS = 16
CHUNK = CHUNK_ROWS * HIDDEN
UNROLL = 8


def kernel(x, table):
    b, n, h = x.shape
    rows = b * n
    rpw = rows // NW
    nchunk = rpw // CHUNK_ROWS

    xf = x.reshape(-1)
    tf = table.reshape(-1)

    mesh = plsc.VectorSubcoreMesh(
        core_axis_name="c", subcore_axis_name="s",
        num_cores=NC, num_subcores=NS)

    @functools.partial(
        pl.kernel,
        out_type=jax.ShapeDtypeStruct((rows * h,), jnp.float32),
        mesh=mesh,
        scratch_types=[
            pltpu.VMEM((2, CHUNK), jnp.float32),
            pltpu.VMEM((2, CHUNK), jnp.float32),
            pltpu.VMEM((2, CHUNK), jnp.float32),
            pltpu.SemaphoreType.DMA((2,)),
            pltpu.SemaphoreType.DMA((2,)),
            pltpu.SemaphoreType.DMA((2,)),
        ],
    )
    def sc_add(x_hbm, t_hbm, o_hbm, xv, tv, ov, semx, semt, semo):
        wid = lax.axis_index("s") * NC + lax.axis_index("c")
        xbase = wid * rpw * h
        tbase = lax.rem(wid * rpw, n) * h

        def start_in(c, buf):
            xoff = xbase + c * CHUNK
            toff = tbase + c * CHUNK
            pltpu.async_copy(
                x_hbm.at[pl.ds(xoff, CHUNK)], xv.at[buf], semx.at[buf])
            pltpu.async_copy(
                t_hbm.at[pl.ds(toff, CHUNK)], tv.at[buf], semt.at[buf])

        def wait_in(c, buf):
            xoff = xbase + c * CHUNK
            toff = tbase + c * CHUNK
            pltpu.make_async_copy(
                x_hbm.at[pl.ds(xoff, CHUNK)], xv.at[buf], semx.at[buf]).wait()
            pltpu.make_async_copy(
                t_hbm.at[pl.ds(toff, CHUNK)], tv.at[buf], semt.at[buf]).wait()

        def start_out(c, buf):
            xoff = xbase + c * CHUNK
            pltpu.async_copy(
                ov.at[buf], o_hbm.at[pl.ds(xoff, CHUNK)], semo.at[buf])

        def wait_out(c, buf):
            xoff = xbase + c * CHUNK
            pltpu.make_async_copy(
                ov.at[buf], o_hbm.at[pl.ds(xoff, CHUNK)], semo.at[buf]).wait()

        start_in(0, 0)
        start_in(1, 1)

        def body(c, carry):
            buf = lax.rem(c, 2)

            wait_in(c, buf)

            @pl.when(c >= 2)
            def _():
                wait_out(c - 2, buf)

            @plsc.parallel_loop(0, CHUNK, step=16, unroll=UNROLL)
            def _add(g):
                ov[buf, pl.ds(g, 16)] = (
                    xv[buf, pl.ds(g, 16)] + tv[buf, pl.ds(g, 16)])

            start_out(c, buf)

            @pl.when(c + 2 < nchunk)
            def _():
                start_in(c + 2, buf)

            return carry

        lax.fori_loop(0, nchunk, body, 0)
        wait_out(nchunk - 2, lax.rem(nchunk - 2, 2))
        wait_out(nchunk - 1, lax.rem(nchunk - 1, 2))

    out = sc_add(xf, tf)
    return out.reshape(b, n, h)


# speedup vs baseline: 3.0461x; 1.0017x over previous
import functools
import jax
import jax.numpy as jnp
from jax import lax
from jax.experimental import pallas as pl
from jax.experimental.pallas import tpu as pltpu
from jax.experimental.pallas import tpu_sc as plsc

HIDDEN = 1024
NC, NS = 2, 16
NW = NC * NS
CHUNK_ROWS = 16
CHUNK = CHUNK_ROWS * HIDDEN
UNROLL = 8


def kernel(x, table):
    b, n, h = x.shape
    rows = b * n
    rpw = rows // NW
    nchunk = rpw // CHUNK_ROWS

    x2 = x.reshape(rows, h)

    mesh = plsc.VectorSubcoreMesh(
        core_axis_name="c", subcore_axis_name="s",
        num_cores=NC, num_subcores=NS)

    @functools.partial(
        pl.kernel,
        out_type=jax.ShapeDtypeStruct((rows, h), jnp.float32),
        mesh=mesh,
        scratch_types=[
            pltpu.VMEM((2, CHUNK_ROWS, h), jnp.float32),
            pltpu.VMEM((2, CHUNK_ROWS, h), jnp.float32),
            pltpu.VMEM((2, CHUNK_ROWS, h), jnp.float32),
            pltpu.SemaphoreType.DMA((2,)),
            pltpu.SemaphoreType.DMA((2,)),
            pltpu.SemaphoreType.DMA((2,)),
        ],
        compiler_params=pltpu.CompilerParams(use_tc_tiling_on_sc=True),
    )
    def sc_add(x_hbm, t_hbm, o_hbm, xv, tv, ov, semx, semt, semo):
        wid = lax.axis_index("s") * NC + lax.axis_index("c")
        rbase = wid * rpw
        tbase = lax.rem(wid * rpw, n)

        def start_in(c, buf):
            pltpu.async_copy(
                x_hbm.at[pl.ds(rbase + c * CHUNK_ROWS, CHUNK_ROWS)],
                xv.at[buf], semx.at[buf])
            pltpu.async_copy(
                t_hbm.at[pl.ds(tbase + c * CHUNK_ROWS, CHUNK_ROWS)],
                tv.at[buf], semt.at[buf])

        def wait_in(c, buf):
            pltpu.make_async_copy(
                x_hbm.at[pl.ds(rbase + c * CHUNK_ROWS, CHUNK_ROWS)],
                xv.at[buf], semx.at[buf]).wait()
            pltpu.make_async_copy(
                t_hbm.at[pl.ds(tbase + c * CHUNK_ROWS, CHUNK_ROWS)],
                tv.at[buf], semt.at[buf]).wait()

        def start_out(c, buf):
            pltpu.async_copy(
                ov.at[buf],
                o_hbm.at[pl.ds(rbase + c * CHUNK_ROWS, CHUNK_ROWS)],
                semo.at[buf])

        def wait_out(c, buf):
            pltpu.make_async_copy(
                ov.at[buf],
                o_hbm.at[pl.ds(rbase + c * CHUNK_ROWS, CHUNK_ROWS)],
                semo.at[buf]).wait()

        start_in(0, 0)
        start_in(1, 1)

        def body(c, carry):
            buf = lax.rem(c, 2)

            wait_in(c, buf)

            @pl.when(c >= 2)
            def _():
                wait_out(c - 2, buf)

            @plsc.parallel_loop(0, CHUNK, step=16, unroll=UNROLL)
            def _add(g):
                r = lax.shift_right_logical(g, 10)
                cc = pl.multiple_of(lax.bitwise_and(g, h - 1), 16)
                ov[buf, r, pl.ds(cc, 16)] = (
                    xv[buf, r, pl.ds(cc, 16)] + tv[buf, r, pl.ds(cc, 16)])

            start_out(c, buf)

            @pl.when(c + 2 < nchunk)
            def _():
                start_in(c + 2, buf)

            return carry

        lax.fori_loop(0, nchunk, body, 0)
        wait_out(nchunk - 2, lax.rem(nchunk - 2, 2))
        wait_out(nchunk - 1, lax.rem(nchunk - 1, 2))

    out = sc_add(x2, table)
    return out.reshape(b, n, h)


# speedup vs baseline: 3.7158x; 1.2198x over previous
import functools
import jax
import jax.numpy as jnp
from jax import lax
from jax.experimental import pallas as pl
from jax.experimental.pallas import tpu as pltpu
from jax.experimental.pallas import tpu_sc as plsc

HIDDEN = 1024
NC, NS = 2, 16
NW = NC * NS
CHUNK_ROWS = 16
CHUNK = CHUNK_ROWS * HIDDEN
UNROLL = 8


def kernel(x, table):
    b, n, h = x.shape
    rows = b * n
    ppw = n // NW
    npc = ppw // CHUNK_ROWS
    nstep = npc * b

    x2 = x.reshape(rows, h)

    mesh = plsc.VectorSubcoreMesh(
        core_axis_name="c", subcore_axis_name="s",
        num_cores=NC, num_subcores=NS)

    @functools.partial(
        pl.kernel,
        out_type=jax.ShapeDtypeStruct((rows, h), jnp.float32),
        mesh=mesh,
        scratch_types=[
            pltpu.VMEM((2, CHUNK_ROWS, h), jnp.float32),
            pltpu.VMEM((2, CHUNK_ROWS, h), jnp.float32),
            pltpu.VMEM((2, CHUNK_ROWS, h), jnp.float32),
            pltpu.SemaphoreType.DMA((2,)),
            pltpu.SemaphoreType.DMA((2,)),
            pltpu.SemaphoreType.DMA((2,)),
        ],
        compiler_params=pltpu.CompilerParams(use_tc_tiling_on_sc=True),
    )
    def sc_add(x_hbm, t_hbm, o_hbm, xv, tv, ov, semx, semt, semo):
        wid = lax.axis_index("s") * NC + lax.axis_index("c")
        pbase = wid * ppw

        def xrow(m):
            pc = lax.shift_right_logical(m, 2)
            be = lax.bitwise_and(m, b - 1)
            return be * n + pbase + pc * CHUNK_ROWS

        def start_x(m, buf):
            pltpu.async_copy(
                x_hbm.at[pl.ds(xrow(m), CHUNK_ROWS)], xv.at[buf],
                semx.at[buf])

        def wait_x(m, buf):
            pltpu.make_async_copy(
                x_hbm.at[pl.ds(xrow(m), CHUNK_ROWS)], xv.at[buf],
                semx.at[buf]).wait()

        def start_t(pc, tbuf):
            pltpu.async_copy(
                t_hbm.at[pl.ds(pbase + pc * CHUNK_ROWS, CHUNK_ROWS)],
                tv.at[tbuf], semt.at[tbuf])

        def wait_t(pc, tbuf):
            pltpu.make_async_copy(
                t_hbm.at[pl.ds(pbase + pc * CHUNK_ROWS, CHUNK_ROWS)],
                tv.at[tbuf], semt.at[tbuf]).wait()

        def start_out(m, buf):
            pltpu.async_copy(
                ov.at[buf], o_hbm.at[pl.ds(xrow(m), CHUNK_ROWS)],
                semo.at[buf])

        def wait_out(m, buf):
            pltpu.make_async_copy(
                ov.at[buf], o_hbm.at[pl.ds(xrow(m), CHUNK_ROWS)],
                semo.at[buf]).wait()

        start_x(0, 0)
        start_x(1, 1)
        start_t(0, 0)

        def body(m, carry):
            buf = lax.rem(m, 2)
            pc = lax.shift_right_logical(m, 2)
            be = lax.bitwise_and(m, b - 1)
            tbuf = lax.rem(pc, 2)

            wait_x(m, buf)

            @pl.when(be == 0)
            def _():
                wait_t(pc, tbuf)

                @pl.when(pc + 1 < npc)
                def _():
                    start_t(pc + 1, lax.rem(pc + 1, 2))

            @pl.when(m >= 2)
            def _():
                wait_out(m - 2, buf)

            @plsc.parallel_loop(0, CHUNK, step=16, unroll=UNROLL)
            def _add(g):
                r = lax.shift_right_logical(g, 10)
                cc = pl.multiple_of(lax.bitwise_and(g, h - 1), 16)
                ov[buf, r, pl.ds(cc, 16)] = (
                    xv[buf, r, pl.ds(cc, 16)] + tv[tbuf, r, pl.ds(cc, 16)])

            start_out(m, buf)

            @pl.when(m + 2 < nstep)
            def _():
                start_x(m + 2, buf)

            return carry

        lax.fori_loop(0, nstep, body, 0)
        wait_out(nstep - 2, lax.rem(nstep - 2, 2))
        wait_out(nstep - 1, lax.rem(nstep - 1, 2))

    out = sc_add(x2, table)
    return out.reshape(b, n, h)
